# baseline (speedup 1.0000x reference)
import jax
import jax.numpy as jnp
from jax import lax
from jax.experimental import pallas as pl
from jax.experimental.pallas import tpu as pltpu
from jax.experimental.pallas import tpu_sc as plsc

NUM_CORES = 2
NUM_SUBCORES = 16
NW = NUM_CORES * NUM_SUBCORES
L = 16
BATCH_SIZE = 16384
BPW = BATCH_SIZE // NW
KD = 32
UN = 1000000
IN_ = 100000
NTC = 7813
CPW = 245
CT = 4
NCHA = 62
MAXC = 7808
SENTV = 0x7FFFFFF0
DUMP = BATCH_SIZE
SW = 128
_DEBUG_PHASE_A = False


def _phase_a(uidx_hbm, u4_hbm, rows_hbm,
             uidx_v, mlist, clist, tbuf, sbuf, rowst, sems, osem):
    cid = lax.axis_index("c")
    sid = lax.axis_index("s")
    wid = cid * NUM_SUBCORES + sid
    lo_col = wid * CPW
    lanes = lax.iota(jnp.int32, L)

    pltpu.sync_copy(uidx_hbm, uidx_v.at[pl.ds(0, BATCH_SIZE)])

    def mstep(j, cnt):
        su = uidx_v[pl.ds(j * L, L)]
        cu = su >> 7
        m = (cu >= lo_col) & (cu < lo_col + CPW)
        pk = ((cu - lo_col) << 21) | ((su & 127) << 14) | (j * L + lanes)
        mi = m.astype(jnp.int32)
        pos = plsc.cumsum(mi) - mi
        plsc.store_scatter(mlist, [cnt + pos], pk, mask=m)
        return cnt + plsc.all_reduce_population_count(m)[0]

    cnt = lax.fori_loop(0, BATCH_SIZE // L, mstep, 0)
    plsc.store_scatter(mlist, [cnt + lanes], jnp.full((L,), SENTV, jnp.int32))
    ngrp = (cnt + L - 1) // L

    def fire(g, b):
        startc = jnp.minimum(lo_col + g * CT, MAXC)
        start = pl.multiple_of(startc * 128, 128)
        for a in range(4):
            pltpu.async_copy(u4_hbm.at[a, :, pl.ds(start, CT * 128)],
                             tbuf.at[b, a], sems.at[b])

    def drain(b):
        pltpu.make_async_copy(u4_hbm.at[:, :, pl.ds(0, CT * 128)],
                              tbuf.at[b], sems.at[b]).wait()

    def drain_row():
        pltpu.make_async_copy(rowst.at[0], rows_hbm.at[pl.ds(0, L)],
                              osem).wait()

    def extract_groups(nloc, width, readrow, gfired):
        npg = (nloc + L - 1) // L

        def pstep(pj, fired):
            pk2 = clist[pl.ds(pj * L, L)]
            tl = pk2 >> 21
            l7 = (pk2 >> 14) & 127
            bpos = pk2 & 16383
            valid = tl < width
            ci = tl * 128 + l7
            slot = fired & 3
            sl = jnp.broadcast_to(slot, (L,)).astype(jnp.int32)

            def fstep(f, c2):
                fa = jnp.broadcast_to(f // 8, (L,)).astype(jnp.int32)
                fs = jnp.broadcast_to(f % 8, (L,)).astype(jnp.int32)
                vals = readrow(fa, fs, ci, valid)
                plsc.store_scatter(
                    rowst,
                    [sl, lanes, jnp.broadcast_to(f, (L,)).astype(jnp.int32)],
                    vals, mask=valid)
                return c2

            lax.fori_loop(0, KD, fstep, 0)
            bpos_d = jnp.where(valid, bpos, DUMP)

            pltpu.async_copy(rowst.at[slot], rows_hbm.at[bpos_d], osem).wait()
            return fired + 1

        return lax.fori_loop(0, npg, pstep, gfired)

    def process(g, b, gfired):
        startc = jnp.minimum(lo_col + g * CT, MAXC)
        s0 = startc - lo_col

        def gstep(gj, ccnt):
            pk = mlist[pl.ds(gj * L, L)]
            tl = pk >> 21
            m2 = (tl >= s0) & (tl < s0 + CT)
            mi = m2.astype(jnp.int32)
            pos = plsc.cumsum(mi) - mi
            plsc.store_scatter(clist, [ccnt + pos], pk - (s0 << 21), mask=m2)
            return ccnt + plsc.all_reduce_population_count(m2)[0]

        ccnt = lax.fori_loop(0, ngrp, gstep, 0)
        plsc.store_scatter(clist, [ccnt + lanes], jnp.full((L,), SENTV, jnp.int32))

        def readrow(fa, fs, ci, valid):
            return plsc.load_gather(
                tbuf, [jnp.broadcast_to(b, (L,)).astype(jnp.int32),
                       fa, fs, ci], mask=valid)

        return extract_groups(ccnt, CT, readrow, gfired)

    def one_chunk(c, gfired):
        fire(c, 0)
        drain(0)
        return process(c, 0, gfired)

    gfired = lax.fori_loop(0, NCHA, one_chunk, 0)

    sg = jnp.where(wid == 31, 1, 0)

    @pl.when(sg == 1)
    def _():
        for a in range(4):
            pltpu.sync_copy(u4_hbm.at[a, :, pl.ds(999936, 64)], sbuf.at[a])

    def straggle(gfired):
        tloc = NTC - 1 - lo_col

        def gstep2(gj, ccnt):
            pk = mlist[pl.ds(gj * L, L)]
            m2 = (pk >> 21) == tloc
            mi = m2.astype(jnp.int32)
            pos = plsc.cumsum(mi) - mi
            plsc.store_scatter(clist, [ccnt + pos], pk - (tloc << 21), mask=m2)
            return ccnt + plsc.all_reduce_population_count(m2)[0]

        ccnt = lax.fori_loop(0, ngrp, gstep2, 0)
        plsc.store_scatter(clist, [ccnt + lanes], jnp.full((L,), SENTV, jnp.int32))

        def readrow2(fa, fs, ci, valid):
            return plsc.load_gather(sbuf, [fa, fs, ci], mask=valid)

        return extract_groups(ccnt, 1, readrow2, gfired)

    gfired = lax.cond(sg == 1, straggle, lambda x: x, gfired)

    del gfired


def _phase_b(iidx_hbm, uidx_hbm, i_hbm, rows_hbm, mu_hbm, ub_hbm, ib_hbm,
             out_hbm, idx_u, idx_i, rows_u, rows_i, bu, bi, mu_v, out_v, sem):
    cid = lax.axis_index("c")
    sid = lax.axis_index("s")
    wid = cid * NUM_SUBCORES + sid
    base = wid * BPW

    pltpu.sync_copy(uidx_hbm.at[pl.ds(base, BPW)], idx_u)
    pltpu.sync_copy(iidx_hbm.at[pl.ds(base, BPW)], idx_i)
    pltpu.sync_copy(mu_hbm, mu_v)
    cps = []
    for q in range(4):
        cps.append(pltpu.async_copy(
            rows_hbm.at[pl.ds(base + q * 128, 128)], rows_u.at[pl.ds(q * 128, 128)],
            sem))
    for g in range(BPW // 128):
        ixu = idx_u.at[pl.ds(g * 128, 128)]
        ixi = idx_i.at[pl.ds(g * 128, 128)]
        dst = pl.ds(g * 128, 128)
        cps.append(pltpu.async_copy(i_hbm.at[ixi], rows_i.at[dst], sem))
        cps.append(pltpu.async_copy(ub_hbm.at[ixu], bu.at[dst], sem))
        cps.append(pltpu.async_copy(ib_hbm.at[ixi], bi.at[dst], sem))
    for cp in cps:
        cp.wait()

    mu_vec = mu_v[...]
    lanes = lax.iota(jnp.int32, L)

    def chunk(c, carry):
        r_ids = c * L + lanes
        acc = mu_vec + plsc.load_gather(bu, [r_ids]) + plsc.load_gather(bi, [r_ids])

        def fstep(k, a2):
            ks = jnp.broadcast_to(k, (L,)).astype(jnp.int32)
            uk = plsc.load_gather(rows_u, [r_ids, ks])
            ik = plsc.load_gather(rows_i, [r_ids, ks])
            return a2 + uk * ik

        acc = lax.fori_loop(0, KD, fstep, acc)
        plsc.store_scatter(out_v, [r_ids], acc)
        return carry

    lax.fori_loop(0, BPW // L, chunk, 0)
    pltpu.sync_copy(out_v, out_hbm.at[pl.ds(base, BPW)])


def kernel(user_indices, item_indices, U_embedding, I_embedding, mu, u_bias, i_bias):
    uidx = user_indices.astype(jnp.int32)
    iidx = item_indices.astype(jnp.int32)
    u4 = U_embedding.T.reshape(KD // 8, 8, UN)
    ub = u_bias.reshape(-1)
    ib = i_bias.reshape(-1)
    mu16 = jnp.broadcast_to(mu.astype(jnp.float32), (L,))

    fa = pl.kernel(
        _phase_a,
        out_type=jax.ShapeDtypeStruct((BATCH_SIZE + 1, SW), jnp.float32),
        mesh=plsc.VectorSubcoreMesh(core_axis_name="c", subcore_axis_name="s"),
        compiler_params=pltpu.CompilerParams(
            needs_layout_passes=False, use_tc_tiling_on_sc=True),
        scratch_types=[
            pltpu.VMEM((BATCH_SIZE + L,), jnp.int32),
            pltpu.VMEM((BATCH_SIZE + L,), jnp.int32),
            pltpu.VMEM((BATCH_SIZE + L,), jnp.int32),
            pltpu.VMEM((2, 4, 8, CT * 128), jnp.float32),
            pltpu.VMEM((4, 8, 64), jnp.float32),
            pltpu.VMEM((4, L, SW), jnp.float32),
            pltpu.SemaphoreType.DMA((2,)),
            pltpu.SemaphoreType.DMA,
        ],
    )
    rows_st = fa(uidx, u4)
    if _DEBUG_PHASE_A:
        rows = rows_st[:BATCH_SIZE, :KD]
        irows = jnp.take(I_embedding, iidx, axis=0)
        return (jnp.sum(rows * irows, axis=1) + mu
                + jnp.take(ub, uidx) + jnp.take(ib, iidx))

    fb = pl.kernel(
        _phase_b,
        out_type=jax.ShapeDtypeStruct((BATCH_SIZE,), jnp.float32),
        mesh=plsc.VectorSubcoreMesh(core_axis_name="c", subcore_axis_name="s"),
        compiler_params=pltpu.CompilerParams(
            needs_layout_passes=False, use_tc_tiling_on_sc=False),
        scratch_types=[
            pltpu.VMEM((BPW,), jnp.int32),
            pltpu.VMEM((BPW,), jnp.int32),
            pltpu.VMEM((BPW, SW), jnp.float32),
            pltpu.VMEM((BPW, KD), jnp.float32),
            pltpu.VMEM((BPW,), jnp.float32),
            pltpu.VMEM((BPW,), jnp.float32),
            pltpu.VMEM((L,), jnp.float32),
            pltpu.VMEM((BPW,), jnp.float32),
            pltpu.SemaphoreType.DMA,
        ],
    )
    return fb(iidx, uidx, I_embedding, rows_st, mu16, ub, ib)


# speedup vs baseline: 1.0083x; 1.0083x over previous
import jax
import jax.numpy as jnp
from jax import lax
from jax.experimental import pallas as pl
from jax.experimental.pallas import tpu as pltpu
from jax.experimental.pallas import tpu_sc as plsc

NUM_CORES = 2
NUM_SUBCORES = 16
NW = NUM_CORES * NUM_SUBCORES
L = 16
BATCH_SIZE = 16384
BPW = BATCH_SIZE // NW
KD = 32
UN = 1000000
IN_ = 100000
NTC = 7813
CPW = 245
CT = 4
NCHA = 62
MAXC = 7808
SENTV = 0x7FFFFFF0
DUMP = BATCH_SIZE
SW = 128
_DEBUG_PHASE_A = False


def _phase_a(uidx_hbm, u4_hbm, rows_hbm,
             uidx_v, mlist, clist, tbuf, sbuf, rowst, sems, osem):
    cid = lax.axis_index("c")
    sid = lax.axis_index("s")
    wid = cid * NUM_SUBCORES + sid
    lo_col = wid * CPW
    lanes = lax.iota(jnp.int32, L)

    pltpu.sync_copy(uidx_hbm, uidx_v.at[pl.ds(0, BATCH_SIZE)])

    def mstep(j, cnt):
        su = uidx_v[pl.ds(j * L, L)]
        cu = su >> 7
        m = (cu >= lo_col) & (cu < lo_col + CPW)
        pk = ((cu - lo_col) << 21) | ((su & 127) << 14) | (j * L + lanes)
        mi = m.astype(jnp.int32)
        pos = plsc.cumsum(mi) - mi
        plsc.store_scatter(mlist, [cnt + pos], pk, mask=m)
        return cnt + plsc.all_reduce_population_count(m)[0]

    cnt = lax.fori_loop(0, BATCH_SIZE // L, mstep, 0)
    plsc.store_scatter(mlist, [cnt + lanes], jnp.full((L,), SENTV, jnp.int32))
    ngrp = (cnt + L - 1) // L

    def fire(g, b):
        startc = jnp.minimum(lo_col + g * CT, MAXC)
        start = pl.multiple_of(startc * 128, 128)
        for a in range(4):
            pltpu.async_copy(u4_hbm.at[a, :, pl.ds(start, CT * 128)],
                             tbuf.at[b, a], sems.at[b])

    def drain(b):
        pltpu.make_async_copy(u4_hbm.at[:, :, pl.ds(0, CT * 128)],
                              tbuf.at[b], sems.at[b]).wait()

    def drain_row():
        pltpu.make_async_copy(rowst.at[0], rows_hbm.at[pl.ds(0, L)],
                              osem).wait()

    def extract_groups(nloc, width, readrow, gfired):
        npg = (nloc + L - 1) // L

        def pstep(pj, fired):
            pk2 = clist[pl.ds(pj * L, L)]
            tl = pk2 >> 21
            l7 = (pk2 >> 14) & 127
            bpos = pk2 & 16383
            valid = tl < width
            ci = tl * 128 + l7
            slot = fired & 3
            sl = jnp.broadcast_to(slot, (L,)).astype(jnp.int32)

            @pl.when(fired >= 4)
            def _():
                pltpu.make_async_copy(rowst.at[0], rows_hbm.at[pl.ds(0, L)],
                                      osem.at[slot]).wait()

            def fstep(f, c2):
                fa = jnp.broadcast_to(f // 8, (L,)).astype(jnp.int32)
                fs = jnp.broadcast_to(f % 8, (L,)).astype(jnp.int32)
                vals = readrow(fa, fs, ci, valid)
                plsc.store_scatter(
                    rowst,
                    [sl, lanes, jnp.broadcast_to(f, (L,)).astype(jnp.int32)],
                    vals, mask=valid)
                return c2

            lax.fori_loop(0, KD, fstep, 0)
            bpos_d = jnp.where(valid, bpos, DUMP)
            pltpu.async_copy(rowst.at[slot], rows_hbm.at[bpos_d], osem.at[slot])
            return fired + 1

        return lax.fori_loop(0, npg, pstep, gfired)

    def process(g, b, gfired):
        startc = jnp.minimum(lo_col + g * CT, MAXC)
        s0 = startc - lo_col

        def gstep(gj, ccnt):
            pk = mlist[pl.ds(gj * L, L)]
            tl = pk >> 21
            m2 = (tl >= s0) & (tl < s0 + CT)
            mi = m2.astype(jnp.int32)
            pos = plsc.cumsum(mi) - mi
            plsc.store_scatter(clist, [ccnt + pos], pk - (s0 << 21), mask=m2)
            return ccnt + plsc.all_reduce_population_count(m2)[0]

        ccnt = lax.fori_loop(0, ngrp, gstep, 0)
        plsc.store_scatter(clist, [ccnt + lanes], jnp.full((L,), SENTV, jnp.int32))

        def readrow(fa, fs, ci, valid):
            return plsc.load_gather(
                tbuf, [jnp.broadcast_to(b, (L,)).astype(jnp.int32),
                       fa, fs, ci], mask=valid)

        return extract_groups(ccnt, CT, readrow, gfired)

    fire(0, 0)
    fire(1, 1)

    def two_chunks(t, gfired):
        c = t * 2
        drain(0)
        gfired = process(c, 0, gfired)

        @pl.when(c + 2 < NCHA)
        def _():
            fire(c + 2, 0)

        drain(1)
        gfired = process(c + 1, 1, gfired)

        @pl.when(c + 3 < NCHA)
        def _():
            fire(c + 3, 1)

        return gfired

    gfired = lax.fori_loop(0, NCHA // 2, two_chunks, 0)

    sg = jnp.where(wid == 31, 1, 0)

    @pl.when(sg == 1)
    def _():
        for a in range(4):
            pltpu.sync_copy(u4_hbm.at[a, :, pl.ds(999936, 64)], sbuf.at[a])

    def straggle(gfired):
        tloc = NTC - 1 - lo_col

        def gstep2(gj, ccnt):
            pk = mlist[pl.ds(gj * L, L)]
            m2 = (pk >> 21) == tloc
            mi = m2.astype(jnp.int32)
            pos = plsc.cumsum(mi) - mi
            plsc.store_scatter(clist, [ccnt + pos], pk - (tloc << 21), mask=m2)
            return ccnt + plsc.all_reduce_population_count(m2)[0]

        ccnt = lax.fori_loop(0, ngrp, gstep2, 0)
        plsc.store_scatter(clist, [ccnt + lanes], jnp.full((L,), SENTV, jnp.int32))

        def readrow2(fa, fs, ci, valid):
            return plsc.load_gather(sbuf, [fa, fs, ci], mask=valid)

        return extract_groups(ccnt, 1, readrow2, gfired)

    gfired = lax.cond(sg == 1, straggle, lambda x: x, gfired)

    for k in range(4):
        @pl.when(gfired > k)
        def _():
            pltpu.make_async_copy(rowst.at[0], rows_hbm.at[pl.ds(0, L)],
                                  osem.at[k]).wait()


def _phase_b(iidx_hbm, uidx_hbm, i_hbm, rows_hbm, mu_hbm, ub_hbm, ib_hbm,
             out_hbm, idx_u, idx_i, rows_u, rows_i, bu, bi, mu_v, out_v, sem):
    cid = lax.axis_index("c")
    sid = lax.axis_index("s")
    wid = cid * NUM_SUBCORES + sid
    base = wid * BPW

    pltpu.sync_copy(uidx_hbm.at[pl.ds(base, BPW)], idx_u)
    pltpu.sync_copy(iidx_hbm.at[pl.ds(base, BPW)], idx_i)
    pltpu.sync_copy(mu_hbm, mu_v)
    cps = []
    for q in range(4):
        cps.append(pltpu.async_copy(
            rows_hbm.at[pl.ds(base + q * 128, 128)], rows_u.at[pl.ds(q * 128, 128)],
            sem))
    for g in range(BPW // 128):
        ixu = idx_u.at[pl.ds(g * 128, 128)]
        ixi = idx_i.at[pl.ds(g * 128, 128)]
        dst = pl.ds(g * 128, 128)
        cps.append(pltpu.async_copy(i_hbm.at[ixi], rows_i.at[dst], sem))
        cps.append(pltpu.async_copy(ub_hbm.at[ixu], bu.at[dst], sem))
        cps.append(pltpu.async_copy(ib_hbm.at[ixi], bi.at[dst], sem))
    for cp in cps:
        cp.wait()

    mu_vec = mu_v[...]
    lanes = lax.iota(jnp.int32, L)

    def chunk(c, carry):
        r_ids = c * L + lanes
        acc = mu_vec + plsc.load_gather(bu, [r_ids]) + plsc.load_gather(bi, [r_ids])

        def fstep(k, a2):
            ks = jnp.broadcast_to(k, (L,)).astype(jnp.int32)
            uk = plsc.load_gather(rows_u, [r_ids, ks])
            ik = plsc.load_gather(rows_i, [r_ids, ks])
            return a2 + uk * ik

        acc = lax.fori_loop(0, KD, fstep, acc)
        plsc.store_scatter(out_v, [r_ids], acc)
        return carry

    lax.fori_loop(0, BPW // L, chunk, 0)
    pltpu.sync_copy(out_v, out_hbm.at[pl.ds(base, BPW)])


def kernel(user_indices, item_indices, U_embedding, I_embedding, mu, u_bias, i_bias):
    uidx = user_indices.astype(jnp.int32)
    iidx = item_indices.astype(jnp.int32)
    u4 = U_embedding.T.reshape(KD // 8, 8, UN)
    ub = u_bias.reshape(-1)
    ib = i_bias.reshape(-1)
    mu16 = jnp.broadcast_to(mu.astype(jnp.float32), (L,))

    fa = pl.kernel(
        _phase_a,
        out_type=jax.ShapeDtypeStruct((BATCH_SIZE + 1, SW), jnp.float32),
        mesh=plsc.VectorSubcoreMesh(core_axis_name="c", subcore_axis_name="s"),
        compiler_params=pltpu.CompilerParams(
            needs_layout_passes=False, use_tc_tiling_on_sc=True),
        scratch_types=[
            pltpu.VMEM((BATCH_SIZE + L,), jnp.int32),
            pltpu.VMEM((BATCH_SIZE + L,), jnp.int32),
            pltpu.VMEM((BATCH_SIZE + L,), jnp.int32),
            pltpu.VMEM((2, 4, 8, CT * 128), jnp.float32),
            pltpu.VMEM((4, 8, 64), jnp.float32),
            pltpu.VMEM((4, L, SW), jnp.float32),
            pltpu.SemaphoreType.DMA((2,)),
            pltpu.SemaphoreType.DMA((4,)),
        ],
    )
    rows_st = fa(uidx, u4)
    if _DEBUG_PHASE_A:
        rows = rows_st[:BATCH_SIZE, :KD]
        irows = jnp.take(I_embedding, iidx, axis=0)
        return (jnp.sum(rows * irows, axis=1) + mu
                + jnp.take(ub, uidx) + jnp.take(ib, iidx))

    fb = pl.kernel(
        _phase_b,
        out_type=jax.ShapeDtypeStruct((BATCH_SIZE,), jnp.float32),
        mesh=plsc.VectorSubcoreMesh(core_axis_name="c", subcore_axis_name="s"),
        compiler_params=pltpu.CompilerParams(
            needs_layout_passes=False, use_tc_tiling_on_sc=False),
        scratch_types=[
            pltpu.VMEM((BPW,), jnp.int32),
            pltpu.VMEM((BPW,), jnp.int32),
            pltpu.VMEM((BPW, SW), jnp.float32),
            pltpu.VMEM((BPW, KD), jnp.float32),
            pltpu.VMEM((BPW,), jnp.float32),
            pltpu.VMEM((BPW,), jnp.float32),
            pltpu.VMEM((L,), jnp.float32),
            pltpu.VMEM((BPW,), jnp.float32),
            pltpu.SemaphoreType.DMA,
        ],
    )
    return fb(iidx, uidx, I_embedding, rows_st, mu16, ub, ib)


# speedup vs baseline: 1.5696x; 1.5566x over previous
import jax
import jax.numpy as jnp
from jax import lax
from jax.experimental import pallas as pl
from jax.experimental.pallas import tpu as pltpu
from jax.experimental.pallas import tpu_sc as plsc

NUM_CORES = 2
NUM_SUBCORES = 16
NW = NUM_CORES * NUM_SUBCORES
LANES = 16
BATCH_SIZE = 16384
BPW = BATCH_SIZE // NW
GCH = 128
NCHUNK = BPW // GCH
KDIM = 32


def _body(uidx_hbm, iidx_hbm, u_hbm, i_hbm, mu_hbm, ub_hbm, ib_hbm, out_hbm,
          idx_u, idx_i, rows_u, rows_i, bu, bi, mu_v, out_v, sem):
    cid = lax.axis_index("c")
    sid = lax.axis_index("s")
    wid = cid * NUM_SUBCORES + sid
    base = wid * BPW

    pltpu.sync_copy(uidx_hbm.at[pl.ds(base, BPW)], idx_u)
    pltpu.sync_copy(iidx_hbm.at[pl.ds(base, BPW)], idx_i)
    pltpu.sync_copy(mu_hbm, mu_v)

    cps = []
    for g in range(NCHUNK):
        ixu = idx_u.at[pl.ds(g * GCH, GCH)]
        ixi = idx_i.at[pl.ds(g * GCH, GCH)]
        dst = pl.ds(g * GCH, GCH)
        cps.append(pltpu.async_copy(u_hbm.at[ixu], rows_u.at[dst], sem))
        cps.append(pltpu.async_copy(i_hbm.at[ixi], rows_i.at[dst], sem))
        cps.append(pltpu.async_copy(ub_hbm.at[ixu], bu.at[dst], sem))
        cps.append(pltpu.async_copy(ib_hbm.at[ixi], bi.at[dst], sem))
    for cp in cps:
        cp.wait()

    mu_vec = mu_v[...]
    lanes = lax.iota(jnp.int32, LANES)

    def chunk(c, carry):
        r_ids = c * LANES + lanes
        acc = mu_vec + plsc.load_gather(bu, [r_ids]) + plsc.load_gather(bi, [r_ids])
        for k in range(KDIM):
            ks = jnp.full((LANES,), k, jnp.int32)
            uk = plsc.load_gather(rows_u, [r_ids, ks])
            ik = plsc.load_gather(rows_i, [r_ids, ks])
            acc = acc + uk * ik
        plsc.store_scatter(out_v, [r_ids], acc)
        return carry

    lax.fori_loop(0, BPW // LANES, chunk, 0)

    pltpu.sync_copy(out_v, out_hbm.at[pl.ds(base, BPW)])


def kernel(user_indices, item_indices, U_embedding, I_embedding, mu, u_bias, i_bias):
    uidx = user_indices.astype(jnp.int32)
    iidx = item_indices.astype(jnp.int32)
    mu16 = jnp.broadcast_to(mu.astype(jnp.float32), (LANES,))
    ub = u_bias.reshape(-1)
    ib = i_bias.reshape(-1)

    f = pl.kernel(
        _body,
        out_type=jax.ShapeDtypeStruct((BATCH_SIZE,), jnp.float32),
        mesh=plsc.VectorSubcoreMesh(core_axis_name="c", subcore_axis_name="s"),
        compiler_params=pltpu.CompilerParams(
            needs_layout_passes=False, use_tc_tiling_on_sc=False),
        scratch_types=[
            pltpu.VMEM((BPW,), jnp.int32),
            pltpu.VMEM((BPW,), jnp.int32),
            pltpu.VMEM((BPW, KDIM), jnp.float32),
            pltpu.VMEM((BPW, KDIM), jnp.float32),
            pltpu.VMEM((BPW,), jnp.float32),
            pltpu.VMEM((BPW,), jnp.float32),
            pltpu.VMEM((LANES,), jnp.float32),
            pltpu.VMEM((BPW,), jnp.float32),
            pltpu.SemaphoreType.DMA,
        ],
    )
    return f(uidx, iidx, U_embedding, I_embedding, mu16, ub, ib)
